# baseline (speedup 1.0000x reference)
import functools

import jax
import jax.numpy as jnp
from jax import lax
from jax.experimental import pallas as pl
from jax.experimental.pallas import tpu as pltpu
from jax.experimental.pallas import tpu_sc as plsc

N = 10000
E = 320000
D = 128

NC = 2
NS = 16
DH = D // NC
CHUNK = 128
NCHUNK = 160
G = 10
NG = NCHUNK // G
NBUF = 5
EPT = E // NS
NPAD = 10240
NT = 10112

_mesh = plsc.VectorSubcoreMesh(core_axis_name="c", subcore_axis_name="s")


@functools.partial(
    pl.kernel,
    mesh=_mesh,
    compiler_params=pltpu.CompilerParams(use_tc_tiling_on_sc=False),
    out_type=jax.ShapeDtypeStruct((NPAD, D), jnp.float32),
    scratch_types=[
        pltpu.VMEM((2, G, CHUNK), jnp.int32),
        pltpu.VMEM((2, G, CHUNK), jnp.int32),
        pltpu.VMEM((NBUF, CHUNK, DH), jnp.float32),
        pltpu.VMEM_SHARED((NT, DH), jnp.float32),
        pltpu.VMEM_SHARED((NPAD, DH), jnp.float32),
        pltpu.SemaphoreType.DMA,
        pltpu.SemaphoreType.DMA,
        pltpu.SemaphoreType.DMA,
        pltpu.SemaphoreType.DMA,
        pltpu.SemaphoreType.DMA,
        pltpu.SemaphoreType.DMA,
        pltpu.SemaphoreType.DMA,
        pltpu.SemaphoreType.DMA,
        pltpu.SemaphoreType.DMA,
        pltpu.SemaphoreType.DMA,
        pltpu.SemaphoreType.DMA,
    ],
)
def _sc_agg(table_hbm, idx_hbm, zeros_hbm, out_hbm,
            src_v, dst_v, rows, table_s, agg_s,
            gs0, gs1, gs2, gs3, gs4, ss0, ss1, ss2, ss3, ss4, isem):
    gsems = (gs0, gs1, gs2, gs3, gs4)
    ssems = (ss0, ss1, ss2, ss3, ss4)
    c = lax.axis_index("c")
    s = lax.axis_index("s")
    rpt = NPAD // NS
    tpt = NT // NS

    pltpu.sync_copy(table_hbm.at[pl.ds(s * tpt, tpt), pl.ds(c * DH, DH)],
                    table_s.at[pl.ds(s * tpt, tpt)])
    pltpu.sync_copy(zeros_hbm.at[pl.ds(s * rpt, rpt)],
                    agg_s.at[pl.ds(s * rpt, rpt)])
    plsc.subcore_barrier()

    def fire_idx(g, slot):
        pltpu.async_copy(idx_hbm.at[0, s, pl.ds(g * G, G)],
                         src_v.at[slot], isem)
        pltpu.async_copy(idx_hbm.at[1, s, pl.ds(g * G, G)],
                         dst_v.at[slot], isem)

    def wait_idx(g, slot):
        pltpu.make_async_copy(idx_hbm.at[0, s, pl.ds(g * G, G)],
                              src_v.at[slot], isem).wait()
        pltpu.make_async_copy(idx_hbm.at[1, s, pl.ds(g * G, G)],
                              dst_v.at[slot], isem).wait()

    def fire_gather(islot, k, b):
        pltpu.async_copy(table_s.at[src_v.at[islot, k]], rows.at[b], gsems[b])

    fire_idx(0, 0)
    wait_idx(0, 0)
    for k in range(NBUF - 1):
        fire_gather(0, k, k)

    def group_body(g, _):
        gslot = g % 2
        nslot = (g + 1) % 2
        for k in range(G):
            jn = k + NBUF - 1
            bn = jn % NBUF
            if k == 1:
                @pl.when(g + 1 < NG)
                def _():
                    fire_idx(g + 1, nslot)
            if k == G - NBUF:
                @pl.when(g + 1 < NG)
                def _():
                    wait_idx(g + 1, nslot)
            @pl.when(g * G + jn < NCHUNK)
            def _():
                @pl.when(g * G + jn >= NBUF)
                def _():
                    pltpu.make_async_copy(
                        rows.at[bn], agg_s.at[dst_v.at[0, 0]],
                        ssems[bn]).wait()
                if jn < G:
                    fire_gather(gslot, jn, bn)
                else:
                    fire_gather(nslot, jn - G, bn)
            b = k % NBUF
            pltpu.make_async_copy(table_s.at[src_v.at[gslot, k]],
                                  rows.at[b], gsems[b]).wait()
            pltpu.async_copy(rows.at[b], agg_s.at[dst_v.at[gslot, k]],
                             ssems[b], add=True)
        return 0

    lax.fori_loop(0, NG, group_body, 0)

    for j in range(NCHUNK - NBUF, NCHUNK):
        b = j % NBUF
        pltpu.make_async_copy(rows.at[b], agg_s.at[dst_v.at[0, 0]],
                              ssems[b]).wait()

    plsc.subcore_barrier()
    pltpu.sync_copy(agg_s.at[pl.ds(s * rpt, rpt)],
                    out_hbm.at[pl.ds(s * rpt, rpt), pl.ds(c * DH, DH)])


def _mlp2_body(x_ref, p_ref, wa_ref, ba_ref, wb_ref, bb_ref, o_ref):
    h = x_ref[...] + p_ref[...]
    h = jnp.maximum(jnp.dot(h, wa_ref[...],
                            preferred_element_type=jnp.float32) + ba_ref[...],
                    0.0)
    h = jnp.maximum(jnp.dot(h, wb_ref[...],
                            preferred_element_type=jnp.float32) + bb_ref[...],
                    0.0)
    o_ref[...] = h


def _mlp3_body(x_ref, p_ref, wa_ref, ba_ref, wb_ref, bb_ref,
               wl_ref, bl_ref, o_ref):
    h = x_ref[...] + p_ref[...]
    h = jnp.maximum(jnp.dot(h, wa_ref[...],
                            preferred_element_type=jnp.float32) + ba_ref[...],
                    0.0)
    h = jnp.maximum(jnp.dot(h, wb_ref[...],
                            preferred_element_type=jnp.float32) + bb_ref[...],
                    0.0)
    o_ref[...] = jnp.dot(h, wl_ref[...],
                         preferred_element_type=jnp.float32) + bl_ref[...]


_BLK = 1000

_row_spec = pl.BlockSpec((_BLK, D), lambda i: (i, 0))
_w_spec = pl.BlockSpec((D, D), lambda i: (0, 0))
_b_spec = pl.BlockSpec((1, D), lambda i: (0, 0))

_mlp2 = pl.pallas_call(
    _mlp2_body,
    grid=(N // _BLK,),
    in_specs=[_row_spec, _row_spec, _w_spec, _b_spec, _w_spec, _b_spec],
    out_specs=_row_spec,
    out_shape=jax.ShapeDtypeStruct((NT, D), jnp.float32),
)

_mlp3 = pl.pallas_call(
    _mlp3_body,
    grid=(N // _BLK,),
    in_specs=[_row_spec, _row_spec, _w_spec, _b_spec, _w_spec, _b_spec,
              _w_spec, _b_spec],
    out_specs=_row_spec,
    out_shape=jax.ShapeDtypeStruct((N, D), jnp.float32),
)


def kernel(x, edge_index, W1a, b1a, W1b, b1b, W2a, b2a, W2b, b2b, Wl, bl):
    idx4 = jnp.pad(edge_index.astype(jnp.int32).reshape(2, NS, EPT),
                   ((0, 0), (0, 0), (0, NCHUNK * CHUNK - EPT)),
                   constant_values=N).reshape(2, NS, NCHUNK, CHUNK)
    zeros = jnp.zeros((NPAD, DH), jnp.float32)
    b1a_, b1b_ = b1a.reshape(1, D), b1b.reshape(1, D)
    b2a_, b2b_, bl_ = b2a.reshape(1, D), b2b.reshape(1, D), bl.reshape(1, D)

    xp = jnp.pad(x, ((0, NT - N), (0, 0)))
    p1 = _sc_agg(xp, idx4, zeros)
    h1 = _mlp2(x, p1, W1a, b1a_, W1b, b1b_)
    p2 = _sc_agg(h1, idx4, zeros)
    out = _mlp3(h1, p2, W2a, b2a_, W2b, b2b_, Wl, bl_)
    return out


# speedup vs baseline: 1.0190x; 1.0190x over previous
import functools

import jax
import jax.numpy as jnp
from jax import lax
from jax.experimental import pallas as pl
from jax.experimental.pallas import tpu as pltpu
from jax.experimental.pallas import tpu_sc as plsc

N = 10000
E = 320000
D = 128

NC = 2
NS = 16
DH = D // NC
CHUNK = 128
NCHUNK = 160
G = 10
NG = NCHUNK // G
NBUF = 5
EPT = E // NS
NPAD = 10240
NT = 10112

_mesh = plsc.VectorSubcoreMesh(core_axis_name="c", subcore_axis_name="s")


@functools.partial(
    pl.kernel,
    mesh=_mesh,
    compiler_params=pltpu.CompilerParams(use_tc_tiling_on_sc=False),
    out_type=jax.ShapeDtypeStruct((NPAD, D), jnp.float32),
    scratch_types=[
        pltpu.VMEM((2, G, CHUNK), jnp.int32),
        pltpu.VMEM((2, G, CHUNK), jnp.int32),
        pltpu.VMEM((NBUF, CHUNK, DH), jnp.float32),
        pltpu.VMEM_SHARED((NT, DH), jnp.float32),
        pltpu.VMEM_SHARED((NPAD, DH), jnp.float32),
        pltpu.SemaphoreType.DMA,
        pltpu.SemaphoreType.DMA,
        pltpu.SemaphoreType.DMA,
        pltpu.SemaphoreType.DMA,
        pltpu.SemaphoreType.DMA,
        pltpu.SemaphoreType.DMA,
        pltpu.SemaphoreType.DMA,
        pltpu.SemaphoreType.DMA,
        pltpu.SemaphoreType.DMA,
        pltpu.SemaphoreType.DMA,
        pltpu.SemaphoreType.DMA,
    ],
)
def _sc_agg(table_hbm, idx_hbm, zeros_hbm, out_hbm,
            src_v, dst_v, rows, table_s, agg_s,
            gs0, gs1, gs2, gs3, gs4, ss0, ss1, ss2, ss3, ss4, isem):
    gsems = (gs0, gs1, gs2, gs3, gs4)
    ssems = (ss0, ss1, ss2, ss3, ss4)
    c = lax.axis_index("c")
    s = lax.axis_index("s")
    rpt = NPAD // NS
    tpt = NT // NS

    pltpu.sync_copy(table_hbm.at[pl.ds(s * tpt, tpt), pl.ds(c * DH, DH)],
                    table_s.at[pl.ds(s * tpt, tpt)])
    pltpu.sync_copy(zeros_hbm.at[pl.ds(s * rpt, rpt)],
                    agg_s.at[pl.ds(s * rpt, rpt)])
    plsc.subcore_barrier()

    def fire_idx(g, slot):
        pltpu.async_copy(idx_hbm.at[0, s, pl.ds(g * G, G)],
                         src_v.at[slot], isem)
        pltpu.async_copy(idx_hbm.at[1, s, pl.ds(g * G, G)],
                         dst_v.at[slot], isem)

    def wait_idx(g, slot):
        pltpu.make_async_copy(idx_hbm.at[0, s, pl.ds(g * G, G)],
                              src_v.at[slot], isem).wait()
        pltpu.make_async_copy(idx_hbm.at[1, s, pl.ds(g * G, G)],
                              dst_v.at[slot], isem).wait()

    def fire_gather(islot, k, b):
        pltpu.async_copy(table_s.at[src_v.at[islot, k]], rows.at[b], gsems[b])

    fire_idx(0, 0)
    wait_idx(0, 0)
    for k in range(NBUF - 1):
        fire_gather(0, k, k)

    def group_body(g, _):
        gslot = g % 2
        nslot = (g + 1) % 2
        for k in range(G):
            jn = k + NBUF - 1
            bn = jn % NBUF
            if k == 1:
                @pl.when(g + 1 < NG)
                def _():
                    fire_idx(g + 1, nslot)
            if k == G - NBUF:
                @pl.when(g + 1 < NG)
                def _():
                    wait_idx(g + 1, nslot)
            @pl.when(g * G + jn < NCHUNK)
            def _():
                @pl.when(g * G + jn >= NBUF)
                def _():
                    pltpu.make_async_copy(
                        rows.at[bn], agg_s.at[dst_v.at[0, 0]],
                        ssems[bn]).wait()
                if jn < G:
                    fire_gather(gslot, jn, bn)
                else:
                    fire_gather(nslot, jn - G, bn)
            b = k % NBUF
            pltpu.make_async_copy(table_s.at[src_v.at[gslot, k]],
                                  rows.at[b], gsems[b]).wait()
            pltpu.async_copy(rows.at[b], agg_s.at[dst_v.at[gslot, k]],
                             ssems[b], add=True)
        return 0

    lax.fori_loop(0, NG, group_body, 0)

    for j in range(NCHUNK - NBUF, NCHUNK):
        b = j % NBUF
        pltpu.make_async_copy(rows.at[b], agg_s.at[dst_v.at[0, 0]],
                              ssems[b]).wait()

    plsc.subcore_barrier()
    pltpu.sync_copy(agg_s.at[pl.ds(s * rpt, rpt)],
                    out_hbm.at[pl.ds(s * rpt, rpt), pl.ds(c * DH, DH)])


def _mlp2_body(x_ref, p_ref, wa_ref, ba_ref, wb_ref, bb_ref, o_ref):
    h = x_ref[...] + p_ref[...]
    h = jnp.maximum(jnp.dot(h, wa_ref[...],
                            preferred_element_type=jnp.float32) + ba_ref[...],
                    0.0)
    h = jnp.maximum(jnp.dot(h, wb_ref[...],
                            preferred_element_type=jnp.float32) + bb_ref[...],
                    0.0)
    o_ref[...] = h


def _mlp3_body(x_ref, p_ref, wa_ref, ba_ref, wb_ref, bb_ref,
               wl_ref, bl_ref, o_ref):
    h = x_ref[...] + p_ref[...]
    h = jnp.maximum(jnp.dot(h, wa_ref[...],
                            preferred_element_type=jnp.float32) + ba_ref[...],
                    0.0)
    h = jnp.maximum(jnp.dot(h, wb_ref[...],
                            preferred_element_type=jnp.float32) + bb_ref[...],
                    0.0)
    o_ref[...] = jnp.dot(h, wl_ref[...],
                         preferred_element_type=jnp.float32) + bl_ref[...]


_BLK = 2000

_row_spec = pl.BlockSpec((_BLK, D), lambda i: (i, 0))
_w_spec = pl.BlockSpec((D, D), lambda i: (0, 0))
_b_spec = pl.BlockSpec((1, D), lambda i: (0, 0))

_mlp2 = pl.pallas_call(
    _mlp2_body,
    grid=(N // _BLK,),
    in_specs=[_row_spec, _row_spec, _w_spec, _b_spec, _w_spec, _b_spec],
    out_specs=_row_spec,
    out_shape=jax.ShapeDtypeStruct((NT, D), jnp.float32),
)

_mlp3 = pl.pallas_call(
    _mlp3_body,
    grid=(N // _BLK,),
    in_specs=[_row_spec, _row_spec, _w_spec, _b_spec, _w_spec, _b_spec,
              _w_spec, _b_spec],
    out_specs=_row_spec,
    out_shape=jax.ShapeDtypeStruct((N, D), jnp.float32),
)


def kernel(x, edge_index, W1a, b1a, W1b, b1b, W2a, b2a, W2b, b2b, Wl, bl):
    idx4 = jnp.pad(edge_index.astype(jnp.int32).reshape(2, NS, EPT),
                   ((0, 0), (0, 0), (0, NCHUNK * CHUNK - EPT)),
                   constant_values=N).reshape(2, NS, NCHUNK, CHUNK)
    zeros = jnp.zeros((NPAD, DH), jnp.float32)
    b1a_, b1b_ = b1a.reshape(1, D), b1b.reshape(1, D)
    b2a_, b2b_, bl_ = b2a.reshape(1, D), b2b.reshape(1, D), bl.reshape(1, D)

    xp = jnp.pad(x, ((0, NT - N), (0, 0)))
    p1 = _sc_agg(xp, idx4, zeros)
    h1 = _mlp2(x, p1, W1a, b1a_, W1b, b1b_)
    p2 = _sc_agg(h1, idx4, zeros)
    out = _mlp3(h1, p2, W2a, b2a_, W2b, b2b_, Wl, bl_)
    return out


# speedup vs baseline: 1.0245x; 1.0054x over previous
import functools

import jax
import jax.numpy as jnp
from jax import lax
from jax.experimental import pallas as pl
from jax.experimental.pallas import tpu as pltpu
from jax.experimental.pallas import tpu_sc as plsc

N = 10000
E = 320000
D = 128

NC = 2
NS = 16
DH = D // NC
CHUNK = 128
NCHUNK = 160
G = 10
NG = NCHUNK // G
NBUF = 5
EPT = E // NS
NPAD = 10240
NT = 10112

_mesh = plsc.VectorSubcoreMesh(core_axis_name="c", subcore_axis_name="s")


@functools.partial(
    pl.kernel,
    mesh=_mesh,
    compiler_params=pltpu.CompilerParams(use_tc_tiling_on_sc=False),
    out_type=jax.ShapeDtypeStruct((NPAD, D), jnp.float32),
    scratch_types=[
        pltpu.VMEM((2, G, CHUNK), jnp.int32),
        pltpu.VMEM((2, G, CHUNK), jnp.int32),
        pltpu.VMEM((NBUF, CHUNK, DH), jnp.float32),
        pltpu.VMEM_SHARED((NT, DH), jnp.float32),
        pltpu.VMEM_SHARED((NPAD, DH), jnp.float32),
        pltpu.SemaphoreType.DMA,
        pltpu.SemaphoreType.DMA,
        pltpu.SemaphoreType.DMA,
        pltpu.SemaphoreType.DMA,
        pltpu.SemaphoreType.DMA,
        pltpu.SemaphoreType.DMA,
        pltpu.SemaphoreType.DMA,
        pltpu.SemaphoreType.DMA,
        pltpu.SemaphoreType.DMA,
        pltpu.SemaphoreType.DMA,
        pltpu.SemaphoreType.DMA,
    ],
)
def _sc_agg(table_hbm, idx_hbm, zeros_hbm, out_hbm,
            src_v, dst_v, rows, table_s, agg_s,
            gs0, gs1, gs2, gs3, gs4, ss0, ss1, ss2, ss3, ss4, isem):
    gsems = (gs0, gs1, gs2, gs3, gs4)
    ssems = (ss0, ss1, ss2, ss3, ss4)
    c = lax.axis_index("c")
    s = lax.axis_index("s")
    rpt = NPAD // NS
    tpt = NT // NS

    pltpu.sync_copy(table_hbm.at[pl.ds(s * tpt, tpt), pl.ds(c * DH, DH)],
                    table_s.at[pl.ds(s * tpt, tpt)])
    pltpu.sync_copy(zeros_hbm.at[pl.ds(s * rpt, rpt)],
                    agg_s.at[pl.ds(s * rpt, rpt)])
    plsc.subcore_barrier()

    def fire_idx(g, slot):
        pltpu.async_copy(idx_hbm.at[0, s, pl.ds(g * G, G)],
                         src_v.at[slot], isem)
        pltpu.async_copy(idx_hbm.at[1, s, pl.ds(g * G, G)],
                         dst_v.at[slot], isem)

    def wait_idx(g, slot):
        pltpu.make_async_copy(idx_hbm.at[0, s, pl.ds(g * G, G)],
                              src_v.at[slot], isem).wait()
        pltpu.make_async_copy(idx_hbm.at[1, s, pl.ds(g * G, G)],
                              dst_v.at[slot], isem).wait()

    def fire_gather(islot, k, b):
        pltpu.async_copy(table_s.at[src_v.at[islot, k]], rows.at[b], gsems[b])

    fire_idx(0, 0)
    wait_idx(0, 0)
    for k in range(NBUF - 1):
        fire_gather(0, k, k)

    def group_body(g, _):
        gslot = g % 2
        nslot = (g + 1) % 2
        for k in range(G):
            jn = k + NBUF - 1
            bn = jn % NBUF
            if k == 1:
                @pl.when(g + 1 < NG)
                def _():
                    fire_idx(g + 1, nslot)
            if k == G - NBUF:
                @pl.when(g + 1 < NG)
                def _():
                    wait_idx(g + 1, nslot)
            @pl.when(g * G + jn < NCHUNK)
            def _():
                @pl.when(g * G + jn >= NBUF)
                def _():
                    pltpu.make_async_copy(
                        rows.at[bn], agg_s.at[dst_v.at[0, 0]],
                        ssems[bn]).wait()
                if jn < G:
                    fire_gather(gslot, jn, bn)
                else:
                    fire_gather(nslot, jn - G, bn)
            b = k % NBUF
            pltpu.make_async_copy(table_s.at[src_v.at[gslot, k]],
                                  rows.at[b], gsems[b]).wait()
            pltpu.async_copy(rows.at[b], agg_s.at[dst_v.at[gslot, k]],
                             ssems[b], add=True)
        return 0

    lax.fori_loop(0, NG, group_body, 0)

    for j in range(NCHUNK - NBUF, NCHUNK):
        b = j % NBUF
        pltpu.make_async_copy(rows.at[b], agg_s.at[dst_v.at[0, 0]],
                              ssems[b]).wait()

    plsc.subcore_barrier()
    pltpu.sync_copy(agg_s.at[pl.ds(s * rpt, rpt)],
                    out_hbm.at[pl.ds(s * rpt, rpt), pl.ds(c * DH, DH)])


def _mlp2_body(x_ref, p_ref, wa_ref, ba_ref, wb_ref, bb_ref, o_ref):
    h = x_ref[...] + p_ref[...]
    h = jnp.maximum(jnp.dot(h, wa_ref[...],
                            preferred_element_type=jnp.float32) + ba_ref[...],
                    0.0)
    h = jnp.maximum(jnp.dot(h, wb_ref[...],
                            preferred_element_type=jnp.float32) + bb_ref[...],
                    0.0)
    o_ref[...] = h


def _mlp3_body(x_ref, p_ref, wa_ref, ba_ref, wb_ref, bb_ref,
               wl_ref, bl_ref, o_ref):
    h = x_ref[...] + p_ref[...]
    h = jnp.maximum(jnp.dot(h, wa_ref[...],
                            preferred_element_type=jnp.float32) + ba_ref[...],
                    0.0)
    h = jnp.maximum(jnp.dot(h, wb_ref[...],
                            preferred_element_type=jnp.float32) + bb_ref[...],
                    0.0)
    o_ref[...] = jnp.dot(h, wl_ref[...],
                         preferred_element_type=jnp.float32) + bl_ref[...]


_BLK = 5000

_row_spec = pl.BlockSpec((_BLK, D), lambda i: (i, 0))
_w_spec = pl.BlockSpec((D, D), lambda i: (0, 0))
_b_spec = pl.BlockSpec((1, D), lambda i: (0, 0))

_mlp2 = pl.pallas_call(
    _mlp2_body,
    grid=(N // _BLK,),
    in_specs=[_row_spec, _row_spec, _w_spec, _b_spec, _w_spec, _b_spec],
    out_specs=_row_spec,
    out_shape=jax.ShapeDtypeStruct((NT, D), jnp.float32),
)

_mlp3 = pl.pallas_call(
    _mlp3_body,
    grid=(N // _BLK,),
    in_specs=[_row_spec, _row_spec, _w_spec, _b_spec, _w_spec, _b_spec,
              _w_spec, _b_spec],
    out_specs=_row_spec,
    out_shape=jax.ShapeDtypeStruct((N, D), jnp.float32),
)


def kernel(x, edge_index, W1a, b1a, W1b, b1b, W2a, b2a, W2b, b2b, Wl, bl):
    idx4 = jnp.pad(edge_index.astype(jnp.int32).reshape(2, NS, EPT),
                   ((0, 0), (0, 0), (0, NCHUNK * CHUNK - EPT)),
                   constant_values=N).reshape(2, NS, NCHUNK, CHUNK)
    zeros = jnp.zeros((NPAD, DH), jnp.float32)
    b1a_, b1b_ = b1a.reshape(1, D), b1b.reshape(1, D)
    b2a_, b2b_, bl_ = b2a.reshape(1, D), b2b.reshape(1, D), bl.reshape(1, D)

    xp = jnp.pad(x, ((0, NT - N), (0, 0)))
    p1 = _sc_agg(xp, idx4, zeros)
    h1 = _mlp2(x, p1, W1a, b1a_, W1b, b1b_)
    p2 = _sc_agg(h1, idx4, zeros)
    out = _mlp3(h1, p2, W2a, b2a_, W2b, b2b_, Wl, bl_)
    return out
